# baseline (speedup 1.0000x reference)
import functools

import jax
import jax.numpy as jnp
from jax import lax
from jax.experimental import pallas as pl
from jax.experimental.pallas import tpu as pltpu
from jax.experimental.pallas import tpu_sc as plsc

B = 4
C = 8
T = 4096
V = 2048
D = 1024

NROWS = B * T
NW = 32
WPB = NW // B
RPW = NROWS // NW
R = 8
NCH = RPW // R
NL = 16
DW = D // 2


def _sc_embed(codes_nat, tables_packed):
    mesh = plsc.VectorSubcoreMesh(core_axis_name="c", subcore_axis_name="s")

    @functools.partial(
        pl.kernel,
        mesh=mesh,
        out_type=jax.ShapeDtypeStruct((NROWS, D), jnp.float32),
        scratch_types=[
            pltpu.VMEM((C, RPW), jnp.int32),
            pltpu.VMEM((C * R, DW), jnp.int32),
            pltpu.VMEM((C * R, DW), jnp.int32),
            pltpu.VMEM((R, D), jnp.float32),
            pltpu.VMEM((R, D), jnp.float32),
            pltpu.SemaphoreType.DMA,
            pltpu.SemaphoreType.DMA,
            pltpu.SemaphoreType.DMA,
            pltpu.SemaphoreType.DMA,
        ],
    )
    def k(codes_hbm, tab_hbm, out_hbm, cvmem, gbuf0, gbuf1, obuf0, obuf1,
          sem0, sem1, ssem0, ssem1):
        gbufs = (gbuf0, gbuf1)
        sems = (sem0, sem1)
        obufs = (obuf0, obuf1)
        ssems = (ssem0, ssem1)
        wid = lax.axis_index("s") * 2 + lax.axis_index("c")
        base = wid * RPW
        bi = wid // WPB
        t0 = (wid % WPB) * RPW
        himask = jnp.full((NL,), -65536, jnp.int32)

        for i in range(C):
            pltpu.sync_copy(codes_hbm.at[bi, i, pl.ds(t0, RPW)], cvmem.at[i])

        def add_off(g, _):
            sl = pl.ds(g * NL, NL)
            for i in range(1, C):
                cvmem[i, sl] = cvmem[i, sl] + (i * V)
            return 0

        lax.fori_loop(0, RPW // NL, add_off, 0)

        def issue(ci, gbuf, sem):
            for i in range(C):
                pltpu.async_copy(
                    tab_hbm.at[cvmem.at[i, pl.ds(ci * R, R)]],
                    gbuf.at[pl.ds(i * R, R)], sem)

        def drain(ci, gbuf, sem):
            for i in range(C):
                pltpu.make_async_copy(
                    tab_hbm.at[cvmem.at[i, pl.ds(ci * R, R)]],
                    gbuf.at[pl.ds(i * R, R)], sem).wait()

        issue(0, gbuf0, sem0)

        def pair(p, _):
            for b in range(2):
                ci = p * 2 + b
                gbuf = gbufs[b]
                obuf = obufs[b]

                @pl.when(ci + 1 < NCH)
                def _():
                    issue(ci + 1, gbufs[1 - b], sems[1 - b])

                drain(ci, gbuf, sems[b])

                @pl.when(ci >= 2)
                def _():
                    pltpu.make_async_copy(
                        obuf, out_hbm.at[pl.ds(base, R)], ssems[b]).wait()

                def reduce_group(g, _):
                    for u in range(2):
                        sl = pl.ds((g * 2 + u) * NL, NL)
                        slh = pl.ds(DW + (g * 2 + u) * NL, NL)
                        for r in range(R):
                            w = gbuf[r, sl]
                            lo = lax.bitcast_convert_type(w << 16,
                                                          jnp.float32)
                            hi = lax.bitcast_convert_type(w & himask,
                                                          jnp.float32)
                            for i in range(1, C):
                                w = gbuf[i * R + r, sl]
                                lo = lo + lax.bitcast_convert_type(
                                    w << 16, jnp.float32)
                                hi = hi + lax.bitcast_convert_type(
                                    w, jnp.float32)
                            obuf[r, sl] = lo
                            obuf[r, slh] = hi
                    return 0

                lax.fori_loop(0, DW // (2 * NL), reduce_group, 0)
                pltpu.async_copy(
                    obuf, out_hbm.at[pl.ds(base + ci * R, R)], ssems[b])
            return 0

        lax.fori_loop(0, NCH // 2, pair, 0)
        for b in range(2):
            pltpu.make_async_copy(
                obufs[b], out_hbm.at[pl.ds(base, R)], ssems[b]).wait()

    return k(codes_nat, tables_packed)


def kernel(codes, tables):
    tf = tables.reshape(C * V, D)
    a = jax.lax.bitcast_convert_type(tf[:, :DW], jnp.int32) + 0x8000
    b = jax.lax.bitcast_convert_type(tf[:, DW:], jnp.int32) + 0x8000
    tables_packed = (jax.lax.shift_right_logical(a, 16) | (b & -65536))
    out = _sc_embed(codes, tables_packed)
    return out.reshape(B, T, D)


# speedup vs baseline: 1.4849x; 1.4849x over previous
import functools

import jax
import jax.numpy as jnp
from jax import lax
from jax.experimental import pallas as pl
from jax.experimental.pallas import tpu as pltpu
from jax.experimental.pallas import tpu_sc as plsc

B = 4
C = 8
T = 4096
V = 2048
D = 1024

NROWS = B * T
NW = 32
WPB = NW // B
RPW = NROWS // NW
R = 8
NCH = RPW // R
NL = 16
DW = D // 2


def _sc_embed(codes_nat, tables_packed):
    mesh = plsc.VectorSubcoreMesh(core_axis_name="c", subcore_axis_name="s")

    @functools.partial(
        pl.kernel,
        mesh=mesh,
        out_type=jax.ShapeDtypeStruct((NROWS, D), jnp.float32),
        scratch_types=[
            pltpu.VMEM((C, RPW), jnp.int32),
            pltpu.VMEM((C * R, DW), jnp.int32),
            pltpu.VMEM((C * R, DW), jnp.int32),
            pltpu.VMEM((R, D), jnp.float32),
            pltpu.VMEM((R, D), jnp.float32),
            pltpu.SemaphoreType.DMA,
            pltpu.SemaphoreType.DMA,
            pltpu.SemaphoreType.DMA,
            pltpu.SemaphoreType.DMA,
        ],
    )
    def k(codes_hbm, tab_hbm, out_hbm, cvmem, gbuf0, gbuf1, obuf0, obuf1,
          sem0, sem1, ssem0, ssem1):
        gbufs = (gbuf0, gbuf1)
        sems = (sem0, sem1)
        obufs = (obuf0, obuf1)
        ssems = (ssem0, ssem1)
        wid = lax.axis_index("s") * 2 + lax.axis_index("c")
        base = wid * RPW
        bi = wid // WPB
        t0 = (wid % WPB) * RPW
        himask = jnp.full((NL,), -65536, jnp.int32)

        for i in range(C):
            pltpu.sync_copy(codes_hbm.at[bi, i, pl.ds(t0, RPW)], cvmem.at[i])

        def add_off(g, _):
            sl = pl.ds(g * NL, NL)
            for i in range(1, C):
                cvmem[i, sl] = cvmem[i, sl] + (i * V)
            return 0

        lax.fori_loop(0, RPW // NL, add_off, 0)

        def issue(ci, gbuf, sem):
            for i in range(C):
                pltpu.async_copy(
                    tab_hbm.at[cvmem.at[i, pl.ds(ci * R, R)]],
                    gbuf.at[pl.ds(i * R, R)], sem)

        def drain(ci, gbuf, sem):
            for i in range(C):
                pltpu.make_async_copy(
                    tab_hbm.at[cvmem.at[i, pl.ds(ci * R, R)]],
                    gbuf.at[pl.ds(i * R, R)], sem).wait()

        issue(0, gbuf0, sem0)

        def pair(p, _):
            for b in range(2):
                ci = p * 2 + b
                gbuf = gbufs[b]
                obuf = obufs[b]

                @pl.when(ci + 1 < NCH)
                def _():
                    issue(ci + 1, gbufs[1 - b], sems[1 - b])

                drain(ci, gbuf, sems[b])

                @pl.when(ci >= 2)
                def _():
                    pltpu.make_async_copy(
                        obuf, out_hbm.at[pl.ds(base, R)], ssems[b]).wait()

                def reduce_group(g, _):
                    sl = pl.ds(g * NL, NL)
                    slh = pl.ds(DW + g * NL, NL)
                    for r in range(R):
                        w = gbuf[r, sl]
                        lo = lax.bitcast_convert_type(w << 16, jnp.float32)
                        hi = lax.bitcast_convert_type(w & himask, jnp.float32)
                        for i in range(1, C):
                            w = gbuf[i * R + r, sl]
                            lo = lo + lax.bitcast_convert_type(w << 16,
                                                               jnp.float32)
                            hi = hi + lax.bitcast_convert_type(w, jnp.float32)
                        obuf[r, sl] = lo
                        obuf[r, slh] = hi
                    return 0

                lax.fori_loop(0, DW // NL, reduce_group, 0)
                pltpu.async_copy(
                    obuf, out_hbm.at[pl.ds(base + ci * R, R)], ssems[b])
            return 0

        lax.fori_loop(0, NCH // 2, pair, 0)
        for b in range(2):
            pltpu.make_async_copy(
                obufs[b], out_hbm.at[pl.ds(base, R)], ssems[b]).wait()

    return k(codes_nat, tables_packed)


def kernel(codes, tables):
    tf = tables.reshape(C * V, D)
    a = jax.lax.bitcast_convert_type(tf[:, :DW], jnp.int32) + 0x8000
    b = jax.lax.bitcast_convert_type(tf[:, DW:], jnp.int32) + 0x8000
    tables_packed = (jax.lax.shift_right_logical(a, 16) | (b & -65536))
    out = _sc_embed(codes, tables_packed)
    return out.reshape(B, T, D)


# speedup vs baseline: 1.5053x; 1.0137x over previous
import functools

import jax
import jax.numpy as jnp
from jax import lax
from jax.experimental import pallas as pl
from jax.experimental.pallas import tpu as pltpu
from jax.experimental.pallas import tpu_sc as plsc

B = 4
C = 8
T = 4096
V = 2048
D = 1024

NROWS = B * T
NW = 32
WPB = NW // B
RPW = NROWS // NW
R = 8
NCH = RPW // R
NL = 16
DW = D // 2


def _sc_embed(codes_nat, tables_packed):
    mesh = plsc.VectorSubcoreMesh(core_axis_name="c", subcore_axis_name="s")

    @functools.partial(
        pl.kernel,
        mesh=mesh,
        out_type=jax.ShapeDtypeStruct((NROWS, D), jnp.float32),
        scratch_types=[
            pltpu.VMEM((C, RPW), jnp.int32),
            pltpu.VMEM((C * R, DW), jnp.int32),
            pltpu.VMEM((C * R, DW), jnp.int32),
            pltpu.VMEM((R, D), jnp.float32),
            pltpu.VMEM((R, D), jnp.float32),
            pltpu.SemaphoreType.DMA,
            pltpu.SemaphoreType.DMA,
            pltpu.SemaphoreType.DMA,
            pltpu.SemaphoreType.DMA,
        ],
    )
    def k(codes_hbm, tab_hbm, out_hbm, cvmem, gbuf0, gbuf1, obuf0, obuf1,
          sem0, sem1, ssem0, ssem1):
        gbufs = (gbuf0, gbuf1)
        sems = (sem0, sem1)
        obufs = (obuf0, obuf1)
        ssems = (ssem0, ssem1)
        wid = lax.axis_index("s") * 2 + lax.axis_index("c")
        base = wid * RPW
        bi = wid // WPB
        t0 = (wid % WPB) * RPW
        himask = jnp.full((NL,), -65536, jnp.int32)

        for i in range(C):
            pltpu.async_copy(codes_hbm.at[bi, i, pl.ds(t0, RPW)],
                             cvmem.at[i], sem1)
        for i in range(C):
            pltpu.make_async_copy(codes_hbm.at[bi, i, pl.ds(t0, RPW)],
                                  cvmem.at[i], sem1).wait()

        def add_off(g, _):
            sl = pl.ds(g * NL, NL)
            for i in range(1, C):
                cvmem[i, sl] = cvmem[i, sl] + (i * V)
            return 0

        lax.fori_loop(0, RPW // NL, add_off, 0)

        def issue(ci, gbuf, sem):
            for i in range(C):
                pltpu.async_copy(
                    tab_hbm.at[cvmem.at[i, pl.ds(ci * R, R)]],
                    gbuf.at[pl.ds(i * R, R)], sem)

        def drain(ci, gbuf, sem):
            for i in range(C):
                pltpu.make_async_copy(
                    tab_hbm.at[cvmem.at[i, pl.ds(ci * R, R)]],
                    gbuf.at[pl.ds(i * R, R)], sem).wait()

        issue(0, gbuf0, sem0)

        def pair(p, _):
            for b in range(2):
                ci = p * 2 + b
                gbuf = gbufs[b]
                obuf = obufs[b]

                @pl.when(ci + 1 < NCH)
                def _():
                    issue(ci + 1, gbufs[1 - b], sems[1 - b])

                drain(ci, gbuf, sems[b])

                @pl.when(ci >= 2)
                def _():
                    pltpu.make_async_copy(
                        obuf, out_hbm.at[pl.ds(base, R)], ssems[b]).wait()

                def reduce_group(g, _):
                    sl = pl.ds(g * NL, NL)
                    slh = pl.ds(DW + g * NL, NL)
                    for r in range(R):
                        w = gbuf[r, sl]
                        lo = lax.bitcast_convert_type(w << 16, jnp.float32)
                        hi = lax.bitcast_convert_type(w & himask, jnp.float32)
                        for i in range(1, C):
                            w = gbuf[i * R + r, sl]
                            lo = lo + lax.bitcast_convert_type(w << 16,
                                                               jnp.float32)
                            hi = hi + lax.bitcast_convert_type(w, jnp.float32)
                        obuf[r, sl] = lo
                        obuf[r, slh] = hi
                    return 0

                lax.fori_loop(0, DW // NL, reduce_group, 0)
                pltpu.async_copy(
                    obuf, out_hbm.at[pl.ds(base + ci * R, R)], ssems[b])
            return 0

        lax.fori_loop(0, NCH // 2, pair, 0)
        for b in range(2):
            pltpu.make_async_copy(
                obufs[b], out_hbm.at[pl.ds(base, R)], ssems[b]).wait()

    return k(codes_nat, tables_packed)


def kernel(codes, tables):
    tf = tables.reshape(C * V, D)
    a = jax.lax.bitcast_convert_type(tf[:, :DW], jnp.int32) + 0x8000
    b = jax.lax.bitcast_convert_type(tf[:, DW:], jnp.int32) + 0x8000
    tables_packed = (jax.lax.shift_right_logical(a, 16) | (b & -65536))
    out = _sc_embed(codes, tables_packed)
    return out.reshape(B, T, D)
